# baseline (speedup 1.0000x reference)
import functools

import jax
import jax.numpy as jnp
from jax import lax
from jax.experimental import pallas as pl
from jax.experimental.pallas import tpu as pltpu
from jax.experimental.pallas import tpu_sc as plsc

_EMB = 32
_B = 16384
_NF = 19
_NC = 2
_NS = 16
_NW = _NC * _NS
_BPW = _B // _NW
_CH = 128
_NCH = _BPW // _CH
_NBUF = 4
_PACK = 4
_GR = 25001
_L = 16


def _body(xt_hbm, *refs):
    tables = refs[:_NF]
    out_hbm = refs[_NF]
    rest = refs[_NF + 1:]
    idx_v = rest[:_NF]
    gidx_v = rest[_NF]
    rows_v = rest[_NF + 1:_NF + 1 + _NBUF]
    oblk_v = rest[_NF + 1 + _NBUF:_NF + 3 + _NBUF]
    isem, gsem, wsem = rest[_NF + 3 + _NBUF:]
    wid = lax.axis_index("s") * _NC + lax.axis_index("c")
    base = wid * _BPW

    idescr = [
        pltpu.async_copy(xt_hbm.at[pl.ds(i * _B + base, _BPW)], idx_v[i], isem)
        for i in range(_NF)
    ]
    for d in idescr:
        d.wait()

    lane = lax.iota(jnp.int32, _L)
    row_stride = lane * 128

    wd = []
    for i in range(_NF):
        def gix(k, _):
            v = idx_v[i][pl.ds(k * _L, _L)]
            gidx_v[pl.ds(k * _L, _L)] = lax.shift_right_logical(v, 2)
            return ()

        lax.fori_loop(0, _BPW // _L, gix, (), unroll=4)

        gd = []
        for c in range(_NCH):
            gd.append(pltpu.async_copy(
                tables[i].at[gidx_v.at[pl.ds(c * _CH, _CH)]],
                rows_v[c % _NBUF], gsem))
            if c >= _NBUF - 1:
                gd[c - _NBUF + 1].wait()
        for c in range(_NCH - _NBUF + 1, _NCH):
            gd[c].wait()

        oblk = oblk_v[i % 2]
        if i >= 2:
            wd[i - 2].wait()

        for c in range(_NCH):
            buf = rows_v[c % _NBUF]

            def ext(t, _):
                g, e = t // _EMB, t % _EMB
                v = idx_v[i][pl.ds(c * _CH + g * _L, _L)]
                off = (row_stride + g * (_L * 128)
                       + lax.shift_left(lax.bitwise_and(v, jnp.int32(3)), 5)
                       + e)
                vals = plsc.load_gather(
                    buf, [lax.shift_right_logical(off, 7),
                          lax.bitwise_and(off, jnp.int32(127))])
                col = jnp.full((_L,), c * _CH, jnp.int32) + g * _L + lane
                plsc.store_scatter(
                    oblk, [jnp.full((_L,), e, jnp.int32), col], vals)
                return ()

            lax.fori_loop(0, (_CH // _L) * _EMB, ext, (), unroll=2)

        wd.append(pltpu.async_copy(
            oblk, out_hbm.at[i, :, pl.ds(base, _BPW)], wsem))
    for d in wd[-2:]:
        d.wait()


_sc_lookup = functools.partial(
    pl.kernel,
    out_type=jax.ShapeDtypeStruct((_NF, _EMB, _B), jnp.float32),
    mesh=plsc.VectorSubcoreMesh(core_axis_name="c", subcore_axis_name="s"),
    compiler_params=pltpu.CompilerParams(needs_layout_passes=False),
    scratch_types=(
        [pltpu.VMEM((_BPW,), jnp.int32) for _ in range(_NF)]
        + [pltpu.VMEM((_BPW,), jnp.int32)]
        + [pltpu.VMEM((_CH, 128), jnp.float32) for _ in range(_NBUF)]
        + [pltpu.VMEM((_EMB, _BPW), jnp.float32) for _ in range(2)]
        + [pltpu.SemaphoreType.DMA] * 3
    ),
)(_body)


def kernel(x, W0, W1, W2, W3, W4, W5, W6, W7, W8, W9, W10, W11, W12, W13,
           W14, W15, W16, W17, W18):
    xt = x.T.reshape(-1)
    packed = []
    for w in (W0, W1, W2, W3, W4, W5, W6, W7, W8, W9, W10, W11, W12, W13,
              W14, W15, W16, W17, W18):
        n = w.shape[0]
        w = jnp.pad(w, ((0, _PACK * _GR - n), (0, 0)))
        packed.append(w.reshape(_GR, _PACK * _EMB))
    out = _sc_lookup(xt, *packed)
    return out.transpose(2, 0, 1)


# speedup vs baseline: 2.1473x; 2.1473x over previous
import functools

import jax
import jax.numpy as jnp
from jax import lax
from jax.experimental import pallas as pl
from jax.experimental.pallas import tpu as pltpu
from jax.experimental.pallas import tpu_sc as plsc

_EMB = 32
_B = 16384
_NF = 19
_NC = 2
_NS = 16
_NW = _NC * _NS
_BPW = _B // _NW
_NBUF = 6


def _body(xt_hbm, *refs):
    tables = refs[:_NF]
    out_hbm = refs[_NF]
    rest = refs[_NF + 1:]
    idx_v = rest[:_NF]
    rows_v = rest[_NF:_NF + _NBUF]
    isem, gsem, wsem = rest[_NF + _NBUF:]
    wid = lax.axis_index("s") * _NC + lax.axis_index("c")
    base = wid * _BPW

    idescr = [
        pltpu.async_copy(xt_hbm.at[pl.ds(i * _B + base, _BPW)], idx_v[i], isem)
        for i in range(_NF)
    ]
    for d in idescr:
        d.wait()

    def gather(i):
        return pltpu.async_copy(tables[i].at[idx_v[i]],
                                rows_v[i % _NBUF], gsem)

    def write(i):
        return pltpu.async_copy(
            rows_v[i % _NBUF],
            out_hbm.at[pl.ds(base, _BPW), pl.ds(i * _EMB, _EMB)], wsem)

    gd = [gather(i) for i in range(_NBUF)]
    wd = []
    for i in range(_NF):
        gd[i].wait()
        wd.append(write(i))
        j = i + _NBUF
        if j < _NF:
            wd[i].wait()
            gd.append(gather(j))
    for i in range(_NF - _NBUF, _NF):
        wd[i].wait()


_sc_lookup = functools.partial(
    pl.kernel,
    out_type=jax.ShapeDtypeStruct((_B, _NF * _EMB), jnp.float32),
    mesh=plsc.VectorSubcoreMesh(core_axis_name="c", subcore_axis_name="s"),
    compiler_params=pltpu.CompilerParams(use_tc_tiling_on_sc=False),
    scratch_types=(
        [pltpu.VMEM((_BPW,), jnp.int32) for _ in range(_NF)]
        + [pltpu.VMEM((_BPW, _EMB), jnp.float32) for _ in range(_NBUF)]
        + [pltpu.SemaphoreType.DMA] * 3
    ),
)(_body)


def kernel(x, W0, W1, W2, W3, W4, W5, W6, W7, W8, W9, W10, W11, W12, W13,
           W14, W15, W16, W17, W18):
    xt = x.T.reshape(-1)
    out = _sc_lookup(xt, W0, W1, W2, W3, W4, W5, W6, W7, W8, W9, W10, W11,
                     W12, W13, W14, W15, W16, W17, W18)
    return out.reshape(_B, _NF, _EMB)
